# Initial kernel scaffold; baseline (speedup 1.0000x reference)
import jax
import jax.numpy as jnp
from jax.experimental import pallas as pl


def kernel(x1, x2, w1, b1, g1, be1, w2, b2, g2, be2):
    raise NotImplementedError("write your pallas kernel here")


# speedup vs baseline: 2.6432x; 2.6432x over previous
import functools

import numpy as np

import jax
import jax.numpy as jnp
from jax.experimental import pallas as pl
from jax.experimental.pallas import tpu as pltpu

_VMEM_LIMIT = 64 * 1024 * 1024


def _conv_stage1(xa_ref, xb_ref, w_ref, b_ref, o_ref, s_ref, pad_ref, *, D, H, Ka):
    pad_ref[...] = jnp.zeros_like(pad_ref)
    pad_ref[1:D + 1, 1:H + 1, :Ka] = xa_ref[0]
    pad_ref[1:D + 1, 1:H + 1, Ka:] = xb_ref[0]
    K = pad_ref.shape[-1]
    Nout = o_ref.shape[-1]
    acc = jnp.zeros((D * H, Nout), jnp.float32)
    for kd in range(3):
        for kh in range(3):
            lhs = pad_ref[kd:kd + D, kh:kh + H, :].reshape(D * H, K)
            acc = acc + jnp.dot(lhs, w_ref[kd, kh],
                                preferred_element_type=jnp.float32)
    acc = acc + b_ref[...]
    o_ref[0] = acc.reshape(D, H, Nout)
    s_ref[0, 0:1, :] = jnp.sum(acc, axis=0, keepdims=True)
    s_ref[0, 1:2, :] = jnp.sum(acc * acc, axis=0, keepdims=True)


def _conv_stage2(y_ref, sc_ref, sh_ref, w_ref, b_ref, o_ref, s_ref, pad_ref,
                 *, D, H):
    pad_ref[...] = jnp.zeros_like(pad_ref)
    pad_ref[1:D + 1, 1:H + 1, :] = jnp.maximum(
        y_ref[0] * sc_ref[...] + sh_ref[...], 0.0).astype(pad_ref.dtype)
    K = pad_ref.shape[-1]
    Nout = o_ref.shape[-1]
    acc = jnp.zeros((D * H, Nout), jnp.float32)
    for kd in range(3):
        for kh in range(3):
            lhs = pad_ref[kd:kd + D, kh:kh + H, :].reshape(D * H, K)
            acc = acc + jnp.dot(lhs, w_ref[kd, kh],
                                preferred_element_type=jnp.float32)
    acc = acc + b_ref[...]
    o_ref[0] = acc.reshape(D, H, Nout)
    s_ref[0, 0:1, :] = jnp.sum(acc, axis=0, keepdims=True)
    s_ref[0, 1:2, :] = jnp.sum(acc * acc, axis=0, keepdims=True)


def _affine_relu(x_ref, sc_ref, sh_ref, o_ref):
    o_ref[...] = jnp.maximum(x_ref[...] * sc_ref[...] + sh_ref[...], 0.0)


def _conv1_call(xa, xb, wcat, brow):
    N, D, H, Ka = xa.shape
    K = wcat.shape[2]
    Nout = wcat.shape[-1]
    body = functools.partial(_conv_stage1, D=D, H=H, Ka=Ka)
    return pl.pallas_call(
        body,
        out_shape=(jax.ShapeDtypeStruct((N, D, H, Nout), jnp.float32),
                   jax.ShapeDtypeStruct((N, 2, Nout), jnp.float32)),
        grid=(N,),
        in_specs=[
            pl.BlockSpec((1, D, H, Ka), lambda n: (n, 0, 0, 0)),
            pl.BlockSpec((1, D, H, K - Ka), lambda n: (n, 0, 0, 0)),
            pl.BlockSpec((3, 3, K, Nout), lambda n: (0, 0, 0, 0)),
            pl.BlockSpec((1, Nout), lambda n: (0, 0)),
        ],
        out_specs=(
            pl.BlockSpec((1, D, H, Nout), lambda n: (n, 0, 0, 0)),
            pl.BlockSpec((1, 2, Nout), lambda n: (n, 0, 0)),
        ),
        scratch_shapes=[pltpu.VMEM((D + 2, H + 2, K), jnp.bfloat16)],
        compiler_params=pltpu.CompilerParams(
            dimension_semantics=("parallel",),
            vmem_limit_bytes=_VMEM_LIMIT),
    )(xa, xb, wcat, brow)


def _conv2_call(y1, scrow, shrow, w2b, brow):
    N, D, H, K = y1.shape
    Nout = w2b.shape[-1]
    body = functools.partial(_conv_stage2, D=D, H=H)
    return pl.pallas_call(
        body,
        out_shape=(jax.ShapeDtypeStruct((N, D, H, Nout), jnp.float32),
                   jax.ShapeDtypeStruct((N, 2, Nout), jnp.float32)),
        grid=(N,),
        in_specs=[
            pl.BlockSpec((1, D, H, K), lambda n: (n, 0, 0, 0)),
            pl.BlockSpec((1, K), lambda n: (0, 0)),
            pl.BlockSpec((1, K), lambda n: (0, 0)),
            pl.BlockSpec((3, 3, K, Nout), lambda n: (0, 0, 0, 0)),
            pl.BlockSpec((1, Nout), lambda n: (0, 0)),
        ],
        out_specs=(
            pl.BlockSpec((1, D, H, Nout), lambda n: (n, 0, 0, 0)),
            pl.BlockSpec((1, 2, Nout), lambda n: (n, 0, 0)),
        ),
        scratch_shapes=[pltpu.VMEM((D + 2, H + 2, K), jnp.bfloat16)],
        compiler_params=pltpu.CompilerParams(
            dimension_semantics=("parallel",),
            vmem_limit_bytes=_VMEM_LIMIT),
    )(y1, scrow, shrow, w2b, brow)


def _affine_relu_call(y2d, scrow, shrow, rb):
    rows, cols = y2d.shape
    return pl.pallas_call(
        _affine_relu,
        out_shape=jax.ShapeDtypeStruct((rows, cols), jnp.float32),
        grid=(rows // rb,),
        in_specs=[
            pl.BlockSpec((rb, cols), lambda r: (r, 0)),
            pl.BlockSpec((1, cols), lambda r: (0, 0)),
            pl.BlockSpec((1, cols), lambda r: (0, 0)),
        ],
        out_specs=pl.BlockSpec((rb, cols), lambda r: (r, 0)),
        compiler_params=pltpu.CompilerParams(
            dimension_semantics=("parallel",),
            vmem_limit_bytes=_VMEM_LIMIT),
    )(y2d, scrow, shrow)


def _interp_matrix(n_in, n_out):
    pos = np.arange(n_out, dtype=np.float64) * (n_in - 1) / (n_out - 1)
    lo = np.clip(np.floor(pos).astype(np.int64), 0, n_in - 2)
    frac = pos - lo
    m = np.zeros((n_out, n_in), np.float32)
    m[np.arange(n_out), lo] += (1.0 - frac).astype(np.float32)
    m[np.arange(n_out), lo + 1] += frac.astype(np.float32)
    return jnp.asarray(m)


def _band_select(W):
    sel = np.zeros((3, W, W), np.float32)
    for kw in range(3):
        for wo in range(W):
            wi = wo + kw - 1
            if 0 <= wi < W:
                sel[kw, wi, wo] = 1.0
    return jnp.asarray(sel)


def _banded(w, sel, W):
    Cin, Cout = w.shape[3], w.shape[4]
    wb = jnp.einsum('abkio,kvw->abviwo', w, sel)
    return wb.reshape(3, 3, W * Cin, W * Cout)


def _bn_affine(stats, gamma, beta, count, eps=1e-5):
    C = gamma.shape[0]
    s = jnp.sum(stats, axis=0).reshape(2, -1, C).sum(axis=1)
    mean = s[0] / count
    var = jnp.maximum(s[1] / count - mean * mean, 0.0)
    scale = gamma * jax.lax.rsqrt(var + eps)
    shift = beta - mean * scale
    return scale, shift


def kernel(x1, x2, w1, b1, g1, be1, w2, b2, g2, be2):
    N, Ca, D, H, W = x2.shape
    Cb = x1.shape[1]
    C1 = w1.shape[-1]
    C2 = w2.shape[-1]
    count = N * D * H * W

    t = x1.transpose(0, 2, 3, 4, 1)
    t = jnp.einsum('ndhwc,Dd->nDhwc', t, _interp_matrix(D // 2, D))
    t = jnp.einsum('nDhwc,Hh->nDHwc', t, _interp_matrix(H // 2, H))
    t = jnp.einsum('nDHwc,Ww->nDHWc', t, _interp_matrix(W // 2, W))
    xb = t.reshape(N, D, H, W * Cb).astype(jnp.bfloat16)
    xa = x2.transpose(0, 2, 3, 4, 1).reshape(N, D, H, W * Ca)
    xa = xa.astype(jnp.bfloat16)

    sel = _band_select(W)
    wcat = jnp.concatenate(
        [_banded(w1[:, :, :, :Ca, :], sel, W),
         _banded(w1[:, :, :, Ca:, :], sel, W)], axis=2).astype(jnp.bfloat16)
    b1row = jnp.tile(b1, W).reshape(1, W * C1)

    y1, st1 = _conv1_call(xa, xb, wcat, b1row)
    sc1, sh1 = _bn_affine(st1, g1, be1, count)

    w2b = _banded(w2, sel, W).astype(jnp.bfloat16)
    b2row = jnp.tile(b2, W).reshape(1, W * C2)
    y2, st2 = _conv2_call(y1,
                          jnp.tile(sc1, W).reshape(1, W * C1),
                          jnp.tile(sh1, W).reshape(1, W * C1),
                          w2b, b2row)
    sc2, sh2 = _bn_affine(st2, g2, be2, count)

    rows = N * D
    cols = H * W * C2
    out = _affine_relu_call(y2.reshape(rows, cols),
                            jnp.tile(sc2, H * W).reshape(1, cols),
                            jnp.tile(sh2, H * W).reshape(1, cols),
                            rb=rows // 8)
    return out.reshape(N, D, H, W, C2).transpose(0, 4, 1, 2, 3)


# speedup vs baseline: 3.6733x; 1.3897x over previous
import functools

import numpy as np

import jax
import jax.numpy as jnp
from jax.experimental import pallas as pl
from jax.experimental.pallas import tpu as pltpu

_VMEM_LIMIT = 64 * 1024 * 1024


def _fill_shifted_slabs(s_ref, D, H):
    s_ref[0, 1:D + 1, 1:H, :] = s_ref[1, 1:D + 1, 0:H - 1, :]
    s_ref[2, 1:D + 1, 0:H - 1, :] = s_ref[1, 1:D + 1, 1:H, :]
    s_ref[0, 1:D + 1, 0:1, :] = jnp.zeros_like(s_ref[0, 1:D + 1, 0:1, :])
    s_ref[2, 1:D + 1, H - 1:H, :] = jnp.zeros_like(s_ref[2, 1:D + 1, H - 1:H, :])
    s_ref[:, 0:1, :, :] = jnp.zeros_like(s_ref[:, 0:1, :, :])
    s_ref[:, D + 1:D + 2, :, :] = jnp.zeros_like(s_ref[:, D + 1:D + 2, :, :])


def _tap_accumulate(s_ref, w_ref, D, H, Nout):
    K = s_ref.shape[-1]
    acc = jnp.zeros((D * H, Nout), jnp.float32)
    for kd in range(3):
        for kh in range(3):
            lhs = s_ref[kh, kd:kd + D].reshape(D * H, K)
            acc = acc + jnp.dot(lhs, w_ref[kd, kh],
                                preferred_element_type=jnp.float32)
    return acc


def _conv_stage1(xa_ref, x1_ref, bw_ref, kdh_ref, w_ref, b_ref,
                 o_ref, s2_ref, s_ref, *, D, H, Ka):
    u1 = jnp.dot(x1_ref[0], bw_ref[...],
                 preferred_element_type=jnp.float32).astype(jnp.bfloat16)
    ub = jnp.dot(kdh_ref[...], u1, preferred_element_type=jnp.float32)
    s_ref[1, 1:D + 1, :, :Ka] = xa_ref[0]
    s_ref[1, 1:D + 1, :, Ka:] = ub.astype(jnp.bfloat16).reshape(D, H, ub.shape[-1])
    _fill_shifted_slabs(s_ref, D, H)

    Nout = o_ref.shape[-1]
    acc = _tap_accumulate(s_ref, w_ref, D, H, Nout) + b_ref[...]
    o_ref[0] = acc.reshape(D, H, Nout)
    s2_ref[0, 0:1, :] = jnp.sum(acc, axis=0, keepdims=True)
    s2_ref[0, 1:2, :] = jnp.sum(acc * acc, axis=0, keepdims=True)


def _conv_stage2(y_ref, sc_ref, sh_ref, w_ref, b_ref,
                 o_ref, s2_ref, s_ref, *, D, H):
    s_ref[1, 1:D + 1, :, :] = jnp.maximum(
        y_ref[0] * sc_ref[...] + sh_ref[...], 0.0).astype(s_ref.dtype)
    _fill_shifted_slabs(s_ref, D, H)

    Nout = o_ref.shape[-1]
    acc = _tap_accumulate(s_ref, w_ref, D, H, Nout) + b_ref[...]
    o_ref[0] = acc.reshape(D, H, Nout)
    s2_ref[0, 0:1, :] = jnp.sum(acc, axis=0, keepdims=True)
    s2_ref[0, 1:2, :] = jnp.sum(acc * acc, axis=0, keepdims=True)


def _affine_relu(x_ref, sc_ref, sh_ref, o_ref):
    o_ref[...] = jnp.maximum(x_ref[...] * sc_ref[...] + sh_ref[...], 0.0)


def _conv1_call(xa, x1t, bw, kdh, wcat, brow):
    N, D, H, Ka = xa.shape
    K = wcat.shape[2]
    Nout = wcat.shape[-1]
    M1, K1 = x1t.shape[1], x1t.shape[2]
    body = functools.partial(_conv_stage1, D=D, H=H, Ka=Ka)
    return pl.pallas_call(
        body,
        out_shape=(jax.ShapeDtypeStruct((N, D, H, Nout), jnp.float32),
                   jax.ShapeDtypeStruct((N, 2, Nout), jnp.float32)),
        grid=(N,),
        in_specs=[
            pl.BlockSpec((1, D, H, Ka), lambda n: (n, 0, 0, 0)),
            pl.BlockSpec((1, M1, K1), lambda n: (n, 0, 0)),
            pl.BlockSpec(bw.shape, lambda n: (0, 0)),
            pl.BlockSpec(kdh.shape, lambda n: (0, 0)),
            pl.BlockSpec((3, 3, K, Nout), lambda n: (0, 0, 0, 0)),
            pl.BlockSpec((1, Nout), lambda n: (0, 0)),
        ],
        out_specs=(
            pl.BlockSpec((1, D, H, Nout), lambda n: (n, 0, 0, 0)),
            pl.BlockSpec((1, 2, Nout), lambda n: (n, 0, 0)),
        ),
        scratch_shapes=[pltpu.VMEM((3, D + 2, H, K), jnp.bfloat16)],
        compiler_params=pltpu.CompilerParams(
            dimension_semantics=("parallel",),
            vmem_limit_bytes=_VMEM_LIMIT),
    )(xa, x1t, bw, kdh, wcat, brow)


def _conv2_call(y1, scrow, shrow, w2b, brow):
    N, D, H, K = y1.shape
    Nout = w2b.shape[-1]
    body = functools.partial(_conv_stage2, D=D, H=H)
    return pl.pallas_call(
        body,
        out_shape=(jax.ShapeDtypeStruct((N, D, H, Nout), jnp.float32),
                   jax.ShapeDtypeStruct((N, 2, Nout), jnp.float32)),
        grid=(N,),
        in_specs=[
            pl.BlockSpec((1, D, H, K), lambda n: (n, 0, 0, 0)),
            pl.BlockSpec((1, K), lambda n: (0, 0)),
            pl.BlockSpec((1, K), lambda n: (0, 0)),
            pl.BlockSpec((3, 3, K, Nout), lambda n: (0, 0, 0, 0)),
            pl.BlockSpec((1, Nout), lambda n: (0, 0)),
        ],
        out_specs=(
            pl.BlockSpec((1, D, H, Nout), lambda n: (n, 0, 0, 0)),
            pl.BlockSpec((1, 2, Nout), lambda n: (n, 0, 0)),
        ),
        scratch_shapes=[pltpu.VMEM((3, D + 2, H, K), jnp.bfloat16)],
        compiler_params=pltpu.CompilerParams(
            dimension_semantics=("parallel",),
            vmem_limit_bytes=_VMEM_LIMIT),
    )(y1, scrow, shrow, w2b, brow)


def _affine_relu_call(y2d, scrow, shrow, rb):
    rows, cols = y2d.shape
    return pl.pallas_call(
        _affine_relu,
        out_shape=jax.ShapeDtypeStruct((rows, cols), jnp.float32),
        grid=(rows // rb,),
        in_specs=[
            pl.BlockSpec((rb, cols), lambda r: (r, 0)),
            pl.BlockSpec((1, cols), lambda r: (0, 0)),
            pl.BlockSpec((1, cols), lambda r: (0, 0)),
        ],
        out_specs=pl.BlockSpec((rb, cols), lambda r: (r, 0)),
        compiler_params=pltpu.CompilerParams(
            dimension_semantics=("parallel",),
            vmem_limit_bytes=_VMEM_LIMIT),
    )(y2d, scrow, shrow)


def _interp_matrix(n_in, n_out):
    pos = np.arange(n_out, dtype=np.float64) * (n_in - 1) / (n_out - 1)
    lo = np.clip(np.floor(pos).astype(np.int64), 0, n_in - 2)
    frac = pos - lo
    m = np.zeros((n_out, n_in), np.float32)
    m[np.arange(n_out), lo] += (1.0 - frac).astype(np.float32)
    m[np.arange(n_out), lo + 1] += frac.astype(np.float32)
    return m


def _band_select(W):
    sel = np.zeros((3, W, W), np.float32)
    for kw in range(3):
        for wo in range(W):
            wi = wo + kw - 1
            if 0 <= wi < W:
                sel[kw, wi, wo] = 1.0
    return jnp.asarray(sel)


def _banded(w, sel, W):
    wb = jnp.einsum('abkio,kvw->abviwo', w, sel)
    return wb.reshape(3, 3, W * w.shape[3], W * w.shape[4])


def _bn_affine(stats, gamma, beta, count, eps=1e-5):
    C = gamma.shape[0]
    s = jnp.sum(stats, axis=0).reshape(2, -1, C).sum(axis=1)
    mean = s[0] / count
    var = jnp.maximum(s[1] / count - mean * mean, 0.0)
    scale = gamma * jax.lax.rsqrt(var + eps)
    shift = beta - mean * scale
    return scale, shift


def kernel(x1, x2, w1, b1, g1, be1, w2, b2, g2, be2):
    N, Ca, D, H, W = x2.shape
    Cb = x1.shape[1]
    C1 = w1.shape[-1]
    C2 = w2.shape[-1]
    count = N * D * H * W

    kdh = jnp.asarray(np.kron(_interp_matrix(D // 2, D),
                              _interp_matrix(H // 2, H)), jnp.bfloat16)
    bw = jnp.asarray(np.kron(_interp_matrix(W // 2, W).T,
                             np.eye(Cb, dtype=np.float32)), jnp.bfloat16)

    xa = x2.transpose(0, 2, 3, 4, 1).reshape(N, D, H, W * Ca).astype(jnp.bfloat16)
    x1t = x1.transpose(0, 2, 3, 4, 1).reshape(
        N, (D // 2) * (H // 2), (W // 2) * Cb).astype(jnp.bfloat16)

    sel = _band_select(W)
    wcat = jnp.concatenate(
        [_banded(w1[:, :, :, :Ca, :], sel, W),
         _banded(w1[:, :, :, Ca:, :], sel, W)], axis=2).astype(jnp.bfloat16)
    b1row = jnp.tile(b1, W).reshape(1, W * C1)

    y1, st1 = _conv1_call(xa, x1t, bw, kdh, wcat, b1row)
    sc1, sh1 = _bn_affine(st1, g1, be1, count)

    w2b = _banded(w2, sel, W).astype(jnp.bfloat16)
    b2row = jnp.tile(b2, W).reshape(1, W * C2)
    y2, st2 = _conv2_call(y1,
                          jnp.tile(sc1, W).reshape(1, W * C1),
                          jnp.tile(sh1, W).reshape(1, W * C1),
                          w2b, b2row)
    sc2, sh2 = _bn_affine(st2, g2, be2, count)

    rows = N * D * H
    cols = W * C2
    out = _affine_relu_call(y2.reshape(rows, cols),
                            jnp.tile(sc2, W).reshape(1, cols),
                            jnp.tile(sh2, W).reshape(1, cols),
                            rb=rows // 8)
    return out.reshape(N, D, H, W, C2).transpose(0, 4, 1, 2, 3)
